# baseline (speedup 1.0000x reference)
import functools

import jax
import jax.numpy as jnp
from jax.experimental import pallas as pl

N_C = 100000
N_P = 20000
N_R = 512
HID = 128
HEADS = 4
DH = HID // HEADS


def _mm_bias(x, W, b, BN=1000):
    N, K = x.shape
    M = W.shape[1]
    assert N % BN == 0, (N, BN)

    def body(x_ref, w_ref, b_ref, o_ref):
        o_ref[...] = jnp.dot(x_ref[...], w_ref[...],
                             preferred_element_type=jnp.float32) + b_ref[...]

    return pl.pallas_call(
        body,
        grid=(N // BN,),
        in_specs=[
            pl.BlockSpec((BN, K), lambda i: (i, 0)),
            pl.BlockSpec((K, M), lambda i: (0, 0)),
            pl.BlockSpec((1, M), lambda i: (0, 0)),
        ],
        out_specs=pl.BlockSpec((BN, M), lambda i: (i, 0)),
        out_shape=jax.ShapeDtypeStruct((N, M), jnp.float32),
    )(x, W, b.reshape(1, M))


def _mm2_bias(xa, Wa, xb, Wb, b, BN=1000):
    N, K = xa.shape
    M = Wa.shape[1]
    assert N % BN == 0

    def body(a_ref, wa_ref, b_ref, wb_ref, bias_ref, o_ref):
        acc = jnp.dot(a_ref[...], wa_ref[...], preferred_element_type=jnp.float32)
        acc += jnp.dot(b_ref[...], wb_ref[...], preferred_element_type=jnp.float32)
        o_ref[...] = acc + bias_ref[...]

    return pl.pallas_call(
        body,
        grid=(N // BN,),
        in_specs=[
            pl.BlockSpec((BN, K), lambda i: (i, 0)),
            pl.BlockSpec((K, M), lambda i: (0, 0)),
            pl.BlockSpec((BN, K), lambda i: (i, 0)),
            pl.BlockSpec((K, M), lambda i: (0, 0)),
            pl.BlockSpec((1, M), lambda i: (0, 0)),
        ],
        out_specs=pl.BlockSpec((BN, M), lambda i: (i, 0)),
        out_shape=jax.ShapeDtypeStruct((N, M), jnp.float32),
    )(xa, Wa, xb, Wb, b.reshape(1, M))


def _gcn(x, ei, W, b, n):
    xw = _mm_bias(x, W, jnp.zeros((W.shape[1],), jnp.float32))
    row, col = ei[0], ei[1]
    deg = jnp.zeros((n,), dtype=x.dtype).at[col].add(1.0)
    dinv = jnp.where(deg > 0, jax.lax.rsqrt(jnp.maximum(deg, 1e-12)), 0.0)
    norm = dinv[row] * dinv[col]
    msg = xw[row] * norm[:, None]
    out = jnp.zeros((n, W.shape[1]), dtype=x.dtype).at[col].add(msg)
    return out + b


def _sage(x_src, x_dst, ei, Wl, bl, Wr, n_dst):
    row, col = ei[0], ei[1]
    s = jnp.zeros((n_dst, x_src.shape[1]), dtype=x_src.dtype).at[col].add(x_src[row])
    cnt = jnp.zeros((n_dst,), dtype=x_src.dtype).at[col].add(1.0)
    mean = s / jnp.maximum(cnt, 1.0)[:, None]
    return _mm2_bias(mean, Wl, x_dst, Wr, bl)


def _prelu(v, a):
    return jnp.where(v >= 0, v, a * v)


def _poi2region(poi_h, poi_to_region, region_adj, p):
    s = jax.ops.segment_sum(poi_h, poi_to_region, num_segments=N_R)
    cnt = jax.ops.segment_sum(jnp.ones((poi_h.shape[0],), poi_h.dtype),
                              poi_to_region, num_segments=N_R)
    reg = s / jnp.maximum(cnt, 1.0)[:, None]
    h = (reg @ p["gat_W"]).reshape(N_R, HEADS, DH)
    asrc = (h * p["gat_asrc"]).sum(-1)
    adst = (h * p["gat_adst"]).sum(-1)
    row, col = region_adj[0], region_adj[1]
    e = jax.nn.leaky_relu(asrc[row] + adst[col], 0.2)
    m = jax.ops.segment_max(e, col, num_segments=N_R)
    m = jnp.where(jnp.isfinite(m), m, 0.0)
    ex = jnp.exp(e - m[col])
    den = jax.ops.segment_sum(ex, col, num_segments=N_R)
    alpha = ex / (den[col] + 1e-16)
    out = jax.ops.segment_sum(alpha[:, :, None] * h[row], col, num_segments=N_R)
    return out.reshape(N_R, HID)


def kernel(x_checkin, x_poi, ei_seq, ei_visits, ei_visited, ei_spatial,
           poi_to_region, region_adjacency, region_area, params):
    p = params
    xc = _mm_bias(x_checkin, p["proj_ck_W"], p["proj_ck_b"])
    xp = _mm_bias(x_poi, p["proj_poi_W"], p["proj_poi_b"])
    for l in ("1", "2"):
        c = (_gcn(xc, ei_seq, p["gcn_seq_W" + l], p["gcn_seq_b" + l], N_C)
             + _sage(xp, xc, ei_visited, p["sage_ved_Wl" + l], p["sage_ved_bl" + l],
                     p["sage_ved_Wr" + l], N_C))
        q = (_sage(xc, xp, ei_visits, p["sage_vis_Wl" + l], p["sage_vis_bl" + l],
                   p["sage_vis_Wr" + l], N_P)
             + _gcn(xp, ei_spatial, p["gcn_sp_W" + l], p["gcn_sp_b" + l], N_P))
        if l == "1":
            xc = _prelu(c, p["prelu_a"])
            xp = _prelu(q, p["prelu_a"])
        else:
            xc, xp = c, q
    region_h = _poi2region(xp, poi_to_region, region_adjacency, p)
    city_h = jax.nn.sigmoid((region_h.T * region_area).sum(axis=1))
    return (xc, xp, region_h, city_h)


# speedup vs baseline: 2.2052x; 2.2052x over previous
import functools

import jax
import jax.numpy as jnp
from jax import lax
from jax.experimental import pallas as pl
from jax.experimental.pallas import tpu as pltpu
from jax.experimental.pallas import tpu_sc as plsc

N_C = 100000
N_P = 20000
N_R = 512
HID = 128
HEADS = 4
DH = HID // HEADS

NCORE = 2
NSUB = 16

_MESH = dict(core_axis_name="c", subcore_axis_name="s",
             num_cores=NCORE, num_subcores=NSUB)
_SC_PARAMS = dict(use_tc_tiling_on_sc=False)


def _ceil_to(x, m):
    return (x + m - 1) // m * m


def _pad_edges(ei, n_pad):
    e = ei.shape[1]
    ep = _ceil_to(e, 4096)
    pad = ep - e
    row = jnp.concatenate([ei[0], jnp.zeros((pad,), jnp.int32)])
    col = jnp.concatenate([ei[1], jnp.full((pad,), n_pad, jnp.int32)])
    return row, col.reshape(ep // 128, 128)


def _zero_fill(buf, d):
    def zb(i, _):
        for k in range(d // 16):
            buf[i, pl.ds(k * 16, 16)] = jnp.zeros((16,), jnp.float32)
        return 0
    lax.fori_loop(0, 128, zb, 0)


def _sc_segsum(tabs, row, col2, n_dst, cw):
    ngr = 128 // cw
    gpc = ngr // NCORE
    v = tabs[0].shape[0]
    e = row.shape[0]
    et = e // NSUB
    nb = et // 128
    n_pad = _ceil_to(n_dst, 128)
    wr = n_pad // NSUB

    def body(*refs):
        tab_r = refs[:ngr]
        row_h, col_h = refs[ngr], refs[ngr + 1]
        outs = refs[ngr + 2:2 * ngr + 2]
        (rowv, colv2, buf0, buf1, zbuf, acc,
         gs0, gs1, ss0, ss1) = refs[2 * ngr + 2:]
        cid = lax.axis_index("c")
        sid = lax.axis_index("s")
        pltpu.sync_copy(row_h.at[pl.ds(sid * et, et)], rowv)
        pltpu.sync_copy(col_h.at[pl.ds(sid * nb, nb)], colv2)
        _zero_fill(zbuf, cw)
        bufs = (buf0, buf1)
        gsems = (gs0, gs1)
        ssems = (ss0, ss1)
        nzf = wr // 128
        rem = wr - nzf * 128

        for grp in range(gpc):
            for half in range(NCORE):
                @pl.when(cid == half)
                def _():
                    cg = half * gpc + grp
                    tab = tab_r[cg]
                    out = outs[cg]

                    def zc(k, _):
                        pltpu.sync_copy(zbuf,
                                        acc.at[pl.ds(sid * wr + k * 128, 128)])
                        return 0
                    lax.fori_loop(0, nzf, zc, 0)
                    if rem:
                        pltpu.sync_copy(
                            zbuf.at[pl.ds(0, rem)],
                            acc.at[pl.ds(sid * wr + nzf * 128, rem)])
                    plsc.subcore_barrier()
                    for b in range(2):
                        pltpu.async_copy(
                            tab.at[rowv.at[pl.ds(b * 128, 128)]],
                            bufs[b], gsems[b])

                    def p2(t, _):
                        for b in range(2):
                            g = t * 2 + b

                            @pl.when(g < nb)
                            def _():
                                pltpu.make_async_copy(
                                    tab.at[rowv.at[pl.ds(g * 128, 128)]],
                                    bufs[b], gsems[b]).wait()
                                pltpu.async_copy(bufs[b], acc.at[colv2.at[g]],
                                                 ssems[b], add=True)

                                @pl.when(g + 2 < nb)
                                def __():
                                    pltpu.make_async_copy(
                                        bufs[b], acc.at[colv2.at[g]],
                                        ssems[b]).wait()
                                    pltpu.async_copy(
                                        tab.at[rowv.at[
                                            pl.ds((g + 2) * 128, 128)]],
                                        bufs[b], gsems[b])
                        return 0

                    lax.fori_loop(0, (nb + 1) // 2, p2, 0)
                    for b in range(2):
                        pltpu.make_async_copy(bufs[b], acc.at[colv2.at[0]],
                                              ssems[b]).wait()
                    plsc.subcore_barrier()
                    pltpu.sync_copy(acc.at[pl.ds(sid * wr, wr)],
                                    out.at[pl.ds(sid * wr, wr)])
                    plsc.subcore_barrier()

    kern = pl.kernel(
        body,
        out_type=[jax.ShapeDtypeStruct((n_pad, cw), jnp.float32)] * ngr,
        mesh=plsc.VectorSubcoreMesh(**_MESH),
        compiler_params=pltpu.CompilerParams(**_SC_PARAMS),
        scratch_types=[
            pltpu.VMEM((et,), jnp.int32),
            pltpu.VMEM((nb, 128), jnp.int32),
            pltpu.VMEM((128, cw), jnp.float32),
            pltpu.VMEM((128, cw), jnp.float32),
            pltpu.VMEM((128, cw), jnp.float32),
            pltpu.VMEM_SHARED((n_pad + 16, cw), jnp.float32),
            pltpu.SemaphoreType.DMA,
            pltpu.SemaphoreType.DMA,
            pltpu.SemaphoreType.DMA,
            pltpu.SemaphoreType.DMA,
        ],
    )
    return kern(*tabs, row, col2)


def _sc_count(col2, n_dst):
    d = 16
    e2 = col2.shape[0] * 128
    et2 = e2 // (NCORE * NSUB)
    nb2 = et2 // 128
    n_pad = _ceil_to(n_dst, 128)
    wr = n_pad // NSUB

    def body(col_h, out_h, colv2, onesb, zbuf, acc, ss):
        cid = lax.axis_index("c")
        sid = lax.axis_index("s")
        pltpu.sync_copy(col_h.at[pl.ds((cid * NSUB + sid) * nb2, nb2)], colv2)
        _zero_fill(zbuf, d)

        def ob(i, _):
            onesb[i, pl.ds(0, 16)] = jnp.ones((16,), jnp.float32)
            return 0
        lax.fori_loop(0, 128, ob, 0)

        nzf = wr // 128
        rem = wr - nzf * 128

        def zc(k, _):
            pltpu.sync_copy(zbuf, acc.at[pl.ds(sid * wr + k * 128, 128)])
            return 0
        lax.fori_loop(0, nzf, zc, 0)
        if rem:
            pltpu.sync_copy(zbuf.at[pl.ds(0, rem)],
                            acc.at[pl.ds(sid * wr + nzf * 128, rem)])
        plsc.subcore_barrier()

        def p2(g, _):
            pltpu.async_copy(onesb, acc.at[colv2.at[g]], ss, add=True)

            @pl.when(g >= 8)
            def _():
                pltpu.make_async_copy(onesb, acc.at[colv2.at[0]], ss).wait()
            return 0

        lax.fori_loop(0, nb2, p2, 0)

        def dr(g, _):
            pltpu.make_async_copy(onesb, acc.at[colv2.at[0]], ss).wait()
            return 0

        lax.fori_loop(0, min(nb2, 8), dr, 0)
        plsc.subcore_barrier()
        pltpu.sync_copy(acc.at[pl.ds(sid * wr, wr)],
                        out_h.at[pl.ds(cid * n_pad + sid * wr, wr)])

    kern = pl.kernel(
        body,
        out_type=jax.ShapeDtypeStruct((2 * n_pad, d), jnp.float32),
        mesh=plsc.VectorSubcoreMesh(**_MESH),
        compiler_params=pltpu.CompilerParams(**_SC_PARAMS),
        scratch_types=[
            pltpu.VMEM((nb2, 128), jnp.int32),
            pltpu.VMEM((128, d), jnp.float32),
            pltpu.VMEM((128, d), jnp.float32),
            pltpu.VMEM_SHARED((n_pad + 16, d), jnp.float32),
            pltpu.SemaphoreType.DMA,
        ],
    )
    out = kern(col2)
    return out[:n_dst, 0] + out[n_pad:n_pad + n_dst, 0]


def _split_specs(n, bn, cw):
    ngr = 128 // cw
    return ([pl.BlockSpec((bn, cw), lambda i: (i, 0))] * ngr,
            [jax.ShapeDtypeStruct((n, cw), jnp.float32)] * ngr)


def _mm_bias(x, w, b, split_cw=None, bn=1000):
    n, k = x.shape
    m = w.shape[1]
    ngr = 0 if split_cw is None else 128 // split_cw

    def body(x_ref, w_ref, b_ref, o_ref, *o_splits):
        res = jnp.dot(x_ref[...], w_ref[...],
                      preferred_element_type=jnp.float32) + b_ref[...]
        o_ref[...] = res
        for g in range(ngr):
            o_splits[g][...] = res[:, g * split_cw:(g + 1) * split_cw]

    out_specs = [pl.BlockSpec((bn, m), lambda i: (i, 0))]
    out_shape = [jax.ShapeDtypeStruct((n, m), jnp.float32)]
    if ngr:
        s, sh = _split_specs(n, bn, split_cw)
        out_specs += s
        out_shape += sh
    res = pl.pallas_call(
        body,
        grid=(n // bn,),
        in_specs=[
            pl.BlockSpec((bn, k), lambda i: (i, 0)),
            pl.BlockSpec((k, m), lambda i: (0, 0)),
            pl.BlockSpec((1, m), lambda i: (0, 0)),
        ],
        out_specs=out_specs,
        out_shape=out_shape,
    )(x, w, b.reshape(1, m))
    return res if ngr else res[0]


def _scale_mm_split(x, s, w, cw, bn=1000):
    n, k = x.shape
    ngr = 128 // cw

    def body(x_ref, s_ref, w_ref, *o_splits):
        res = jnp.dot(x_ref[...] * s_ref[...], w_ref[...],
                      preferred_element_type=jnp.float32)
        for g in range(ngr):
            o_splits[g][...] = res[:, g * cw:(g + 1) * cw]

    specs, shapes = _split_specs(n, bn, cw)
    return pl.pallas_call(
        body,
        grid=(n // bn,),
        in_specs=[
            pl.BlockSpec((bn, k), lambda i: (i, 0)),
            pl.BlockSpec((bn, 1), lambda i: (i, 0)),
            pl.BlockSpec((k, 128), lambda i: (0, 0)),
        ],
        out_specs=specs,
        out_shape=shapes,
    )(x, s, w)


def _layer_combine(a_parts, acc_parts, x, wl, wr, dinv, rcnt, bias, alpha,
                   prelu, split_cw=None, bn=1000):
    n = x.shape[0]
    na = len(a_parts)
    nacc = len(acc_parts)
    nb2 = acc_parts[0].shape[0] // bn
    ngr = 0 if split_cw is None else 128 // split_cw
    acw = 128 // na
    ccw = 128 // nacc

    def body(*refs):
        a_refs = refs[:na]
        acc_refs = refs[na:na + nacc]
        (x_ref, wl_ref, wr_ref, d_ref, r_ref, b_ref, al_ref) = \
            refs[na + nacc:na + nacc + 7]
        o_ref = refs[na + nacc + 7]
        o_splits = refs[na + nacc + 8:]
        a = jnp.concatenate([r[...] for r in a_refs], axis=1)
        acc = jnp.concatenate([r[...] for r in acc_refs], axis=1)
        mean = acc * r_ref[...]
        c = d_ref[...] * a + b_ref[...]
        c += jnp.dot(mean, wl_ref[...], preferred_element_type=jnp.float32)
        c += jnp.dot(x_ref[...], wr_ref[...],
                     preferred_element_type=jnp.float32)
        if prelu:
            c = jnp.where(c >= 0, c, al_ref[0, 0] * c)
        o_ref[...] = c
        for g in range(ngr):
            o_splits[g][...] = c[:, g * split_cw:(g + 1) * split_cw]

    in_specs = (
        [pl.BlockSpec((bn, acw), lambda i: (i, 0))] * na
        + [pl.BlockSpec((bn, ccw), lambda i: (jnp.minimum(i, nb2 - 1), 0))] * nacc
        + [
            pl.BlockSpec((bn, HID), lambda i: (i, 0)),
            pl.BlockSpec((HID, HID), lambda i: (0, 0)),
            pl.BlockSpec((HID, HID), lambda i: (0, 0)),
            pl.BlockSpec((bn, 1), lambda i: (i, 0)),
            pl.BlockSpec((bn, 1), lambda i: (i, 0)),
            pl.BlockSpec((1, HID), lambda i: (0, 0)),
            pl.BlockSpec((1, 1), lambda i: (0, 0)),
        ]
    )
    out_specs = [pl.BlockSpec((bn, HID), lambda i: (i, 0))]
    out_shape = [jax.ShapeDtypeStruct((n, HID), jnp.float32)]
    if ngr:
        s, sh = _split_specs(n, bn, split_cw)
        out_specs += s
        out_shape += sh
    res = pl.pallas_call(
        body,
        grid=(n // bn,),
        in_specs=in_specs,
        out_specs=out_specs,
        out_shape=out_shape,
    )(*a_parts, *acc_parts, x, wl, wr, dinv, rcnt, bias.reshape(1, HID),
      alpha.reshape(1, 1))
    return res


def _region_segsum(p2r, xp, be=1000):
    n = xp.shape[0]
    nb = n // be

    def body(p_ref, x_ref, os_ref, oc_ref):
        i = pl.program_id(0)
        rows = lax.broadcasted_iota(jnp.int32, (N_R, be), 0)
        oh = (rows == p_ref[0]).astype(jnp.float32)
        s = jnp.dot(oh, x_ref[...], preferred_element_type=jnp.float32)
        cnt = jnp.sum(oh, axis=1, keepdims=True)

        @pl.when(i == 0)
        def _():
            os_ref[...] = s
            oc_ref[...] = jnp.broadcast_to(cnt, (N_R, HID))

        @pl.when(i != 0)
        def _():
            os_ref[...] += s
            oc_ref[...] += jnp.broadcast_to(cnt, (N_R, HID))

    return pl.pallas_call(
        body,
        grid=(nb,),
        in_specs=[
            pl.BlockSpec((1, 1, be), lambda i: (i, 0, 0)),
            pl.BlockSpec((be, HID), lambda i: (i, 0)),
        ],
        out_specs=[
            pl.BlockSpec((N_R, HID), lambda i: (0, 0)),
            pl.BlockSpec((N_R, HID), lambda i: (0, 0)),
        ],
        out_shape=[
            jax.ShapeDtypeStruct((N_R, HID), jnp.float32),
            jax.ShapeDtypeStruct((N_R, HID), jnp.float32),
        ],
    )(p2r.reshape(nb, 1, be), xp)


def _region_gat(rsum, rcntb, w, asrc, adst, erow, ecol, area):
    ne = erow.shape[0]

    def body(s_ref, c_ref, w_ref, as_ref, ad_ref, re_ref, ce_ref, ar_ref,
             oh_ref, oc_ref):
        cnt = c_ref[:, 0:1]
        mean = s_ref[...] / jnp.maximum(cnt, 1.0)
        h = jnp.dot(mean, w_ref[...], preferred_element_type=jnp.float32)
        iot = lax.broadcasted_iota(jnp.int32, (ne, N_R), 1)
        oh_r = (re_ref[...] == iot).astype(jnp.float32)
        oh_c = (ce_ref[...] == iot).astype(jnp.float32)
        parts = []
        for hd in range(HEADS):
            hh = h[:, hd * DH:(hd + 1) * DH]
            asn = jnp.sum(hh * as_ref[hd:hd + 1, :], axis=1, keepdims=True)
            adn = jnp.sum(hh * ad_ref[hd:hd + 1, :], axis=1, keepdims=True)
            e = (jnp.dot(oh_r, asn, preferred_element_type=jnp.float32)
                 + jnp.dot(oh_c, adn, preferred_element_type=jnp.float32))
            e = jnp.where(e >= 0, e, 0.2 * e)
            big = jnp.where(oh_c > 0, e, -jnp.inf)
            mh = jnp.max(big, axis=0, keepdims=True)
            mh = jnp.where(mh == -jnp.inf, 0.0, mh)
            mcol = lax.dot_general(oh_c, mh, (((1,), (1,)), ((), ())),
                                   preferred_element_type=jnp.float32)
            ex = jnp.exp(e - mcol)
            den = lax.dot_general(oh_c, ex, (((0,), (0,)), ((), ())),
                                  preferred_element_type=jnp.float32)
            den_e = jnp.dot(oh_c, den, preferred_element_type=jnp.float32)
            alpha = ex / (den_e + 1e-16)
            hrow = jnp.dot(oh_r, hh, preferred_element_type=jnp.float32)
            outh = lax.dot_general(oh_c, alpha * hrow, (((0,), (0,)), ((), ())),
                                   preferred_element_type=jnp.float32)
            parts.append(outh)
        regh = jnp.concatenate(parts, axis=1)
        oh_ref[...] = regh
        z = jnp.dot(ar_ref[...], regh, preferred_element_type=jnp.float32)
        oc_ref[...] = 1.0 / (1.0 + jnp.exp(-z))

    return pl.pallas_call(
        body,
        in_specs=[
            pl.BlockSpec((N_R, HID), lambda: (0, 0)),
            pl.BlockSpec((N_R, HID), lambda: (0, 0)),
            pl.BlockSpec((HID, HID), lambda: (0, 0)),
            pl.BlockSpec((HEADS, DH), lambda: (0, 0)),
            pl.BlockSpec((HEADS, DH), lambda: (0, 0)),
            pl.BlockSpec((ne, 1), lambda: (0, 0)),
            pl.BlockSpec((ne, 1), lambda: (0, 0)),
            pl.BlockSpec((1, N_R), lambda: (0, 0)),
        ],
        out_specs=[
            pl.BlockSpec((N_R, HID), lambda: (0, 0)),
            pl.BlockSpec((1, HID), lambda: (0, 0)),
        ],
        out_shape=[
            jax.ShapeDtypeStruct((N_R, HID), jnp.float32),
            jax.ShapeDtypeStruct((1, HID), jnp.float32),
        ],
    )(rsum, rcntb, w, asrc, adst, erow.reshape(ne, 1), ecol.reshape(ne, 1),
      area.reshape(1, N_R))


def kernel(x_checkin, x_poi, ei_seq, ei_visits, ei_visited, ei_spatial,
           poi_to_region, region_adjacency, region_area, params):
    p = params
    npad_c = _ceil_to(N_C, 128)
    npad_p = _ceil_to(N_P, 128)
    row_seq, col_seq = _pad_edges(ei_seq, npad_c)
    row_ved, col_ved = _pad_edges(ei_visited, npad_p)
    row_vis, col_vis = _pad_edges(ei_visits, npad_p)
    row_sp, col_sp = _pad_edges(ei_spatial, npad_p)

    cnt_seq = _sc_count(col_seq, N_C)
    cnt_ved = _sc_count(col_ved, N_P)
    cnt_vis = _sc_count(col_vis, N_P)
    cnt_sp = _sc_count(col_sp, N_P)

    def dinv_of(cnt):
        return jnp.where(cnt > 0, lax.rsqrt(jnp.maximum(cnt, 1e-12)),
                         0.0).reshape(-1, 1)

    dinv_seq = dinv_of(cnt_seq)
    dinv_sp = dinv_of(cnt_sp)
    rcnt_ved = jnp.concatenate(
        [1.0 / jnp.maximum(cnt_ved, 1.0), jnp.zeros((N_C - N_P,), jnp.float32)]
    ).reshape(-1, 1)
    rcnt_vis = (1.0 / jnp.maximum(cnt_vis, 1.0)).reshape(-1, 1)

    xc, *xc_sp64 = _mm_bias(x_checkin, p["proj_ck_W"], p["proj_ck_b"],
                            split_cw=64)
    xp, *xp_sp64 = _mm_bias(x_poi, p["proj_poi_W"], p["proj_poi_b"],
                            split_cw=64)
    alpha = p["prelu_a"]

    for l in ("1", "2"):
        ys_seq = _scale_mm_split(xc, dinv_seq, p["gcn_seq_W" + l], cw=16)
        a_seq = _sc_segsum(ys_seq, row_seq, col_seq, N_C, cw=16)
        acc_ved = _sc_segsum(xp_sp64, row_ved, col_ved, N_P, cw=64)
        ys_sp = _scale_mm_split(xp, dinv_sp, p["gcn_sp_W" + l], cw=64)
        a_sp = _sc_segsum(ys_sp, row_sp, col_sp, N_P, cw=64)
        acc_vis = _sc_segsum(xc_sp64, row_vis, col_vis, N_P, cw=64)
        prelu = l == "1"
        split = 64 if prelu else None
        c_out = _layer_combine(
            a_seq, acc_ved, xc, p["sage_ved_Wl" + l], p["sage_ved_Wr" + l],
            dinv_seq, rcnt_ved, p["gcn_seq_b" + l] + p["sage_ved_bl" + l],
            alpha, prelu, split_cw=split)
        q_out = _layer_combine(
            a_sp, acc_vis, xp, p["sage_vis_Wl" + l], p["sage_vis_Wr" + l],
            dinv_sp, rcnt_vis, p["gcn_sp_b" + l] + p["sage_vis_bl" + l],
            alpha, prelu, split_cw=split)
        xc, xc_sp64 = c_out[0], c_out[1:]
        xp, xp_sp64 = q_out[0], q_out[1:]

    rsum, rcntb = _region_segsum(poi_to_region, xp)
    region_h, city = _region_gat(rsum, rcntb, p["gat_W"], p["gat_asrc"],
                                 p["gat_adst"], region_adjacency[0],
                                 region_adjacency[1], region_area)
    return (xc, xp, region_h, city.reshape(HID))
